# Initial kernel scaffold; baseline (speedup 1.0000x reference)
import jax
import jax.numpy as jnp
from jax.experimental import pallas as pl


def kernel(timestep, pe_matrix):
    raise NotImplementedError("write your pallas kernel here")


# speedup vs baseline: 1.4509x; 1.4509x over previous
import functools

import jax
import jax.numpy as jnp
from jax import lax
from jax.experimental import pallas as pl
from jax.experimental.pallas import tpu as pltpu
from jax.experimental.pallas import tpu_sc as plsc

DIM = 1024
MAX_T = 1000
BATCH = 16384

NC = 2
NS = 16
NW = NC * NS
BPW = BATCH // NW
CHUNK = 32
NCHUNK = BPW // CHUNK


def _gather_body(table_hbm, idx_hbm, out_hbm, idx_v, buf0, buf1,
                 gsem0, gsem1, osem0, osem1):
    cid = lax.axis_index("c")
    sid = lax.axis_index("s")
    wid = sid * NC + cid
    base = wid * BPW

    pltpu.sync_copy(idx_hbm.at[wid], idx_v)

    bufs = (buf0, buf1)
    gsems = (gsem0, gsem1)
    osems = (osem0, osem1)
    g_handles = [None] * NCHUNK
    o_handles = [None, None]

    for j in range(NCHUNK + 1):
        b = j % 2
        if j < NCHUNK:
            if j >= 2:
                o_handles[b].wait()
            g_handles[j] = pltpu.async_copy(
                table_hbm.at[idx_v.at[j]], bufs[b], gsems[b])
        if j >= 1:
            pb = (j - 1) % 2
            g_handles[j - 1].wait()
            o_handles[pb] = pltpu.async_copy(
                bufs[pb], out_hbm.at[pl.ds(base + (j - 1) * CHUNK, CHUNK)],
                osems[pb])
    o_handles[0].wait()
    o_handles[1].wait()


@jax.jit
def _pe_lookup(table, idx3):
    mesh = plsc.VectorSubcoreMesh(core_axis_name="c", subcore_axis_name="s")
    k = functools.partial(
        pl.kernel,
        mesh=mesh,
        out_type=jax.ShapeDtypeStruct((BATCH, DIM), jnp.float32),
        scratch_types=[
            pltpu.VMEM((NCHUNK, CHUNK), jnp.int32),
            pltpu.VMEM((CHUNK, DIM), jnp.float32),
            pltpu.VMEM((CHUNK, DIM), jnp.float32),
            pltpu.SemaphoreType.DMA,
            pltpu.SemaphoreType.DMA,
            pltpu.SemaphoreType.DMA,
            pltpu.SemaphoreType.DMA,
        ],
    )(_gather_body)
    return k(table, idx3)


def kernel(timestep, pe_matrix):
    table = pe_matrix.reshape(MAX_T, DIM)
    idx3 = timestep.astype(jnp.int32).reshape(NW, NCHUNK, CHUNK)
    out = _pe_lookup(table, idx3)
    return out.reshape(BATCH, 1, DIM)
